# baseline (speedup 1.0000x reference)
import functools

import jax
import jax.numpy as jnp
from jax import lax
from jax.experimental import pallas as pl
from jax.experimental.pallas import tpu as pltpu
from jax.experimental.pallas import tpu_sc as plsc

VOCAB = 1000000
EMBED_DIM = 64
BATCH = 16384

_INFO = plsc.get_sparse_core_info()
_NC = _INFO.num_cores
_NS = _INFO.num_subcores
_NW = _NC * _NS
_BPW = BATCH // _NW
_CHUNK = 128
_NCHUNK = _BPW // _CHUNK
_GROUPS = _BPW // 16


def _sgns_body(c_hbm, w_hbm, c_emb_hbm, w_emb_hbm, out_hbm,
               idx_c, idx_w, cv, wv, pscr, out_v, sem):
    wid = lax.axis_index("s") * _NC + lax.axis_index("c")
    base = wid * _BPW

    for t in range(_NCHUNK):
        pltpu.sync_copy(c_hbm.at[pl.ds(base + t * _CHUNK, _CHUNK)], idx_c.at[t])
        pltpu.sync_copy(w_hbm.at[pl.ds(base + t * _CHUNK, _CHUNK)], idx_w.at[t])

    copies = []
    for t in range(_NCHUNK):
        copies.append(pltpu.async_copy(
            c_emb_hbm.at[idx_c.at[t]], cv.at[pl.ds(t * _CHUNK, _CHUNK)], sem))
        copies.append(pltpu.async_copy(
            w_emb_hbm.at[idx_w.at[t]], wv.at[pl.ds(t * _CHUNK, _CHUNK)], sem))
    for cp in copies:
        cp.wait()

    lanes = lax.iota(jnp.int32, 16)

    def group(g, carry):
        rbase = g * 16
        for r in range(16):
            row = rbase + r
            acc = cv[row, pl.ds(0, 16)] * wv[row, pl.ds(0, 16)]
            for k in range(1, EMBED_DIM // 16):
                acc = acc + cv[row, pl.ds(k * 16, 16)] * wv[row, pl.ds(k * 16, 16)]
            pscr[r, :] = acc
        tot = plsc.load_gather(pscr, [lanes, jnp.full((16,), 0, jnp.int32)])
        for j in range(1, 16):
            tot = tot + plsc.load_gather(
                pscr, [lanes, jnp.full((16,), j, jnp.int32)])
        prob = 1.0 / (1.0 + jnp.exp(-tot))
        out_v[pl.ds(rbase, 16)] = prob
        return carry

    lax.fori_loop(0, _GROUPS, group, 0)

    pltpu.sync_copy(out_v, out_hbm.at[pl.ds(base, _BPW)])


@jax.jit
def _sgns(c, w, c_embeds, w_embeds):
    mesh = plsc.VectorSubcoreMesh(core_axis_name="c", subcore_axis_name="s")
    run = functools.partial(
        pl.kernel,
        mesh=mesh,
        compiler_params=pltpu.CompilerParams(
            needs_layout_passes=False, use_tc_tiling_on_sc=False),
        out_type=jax.ShapeDtypeStruct((BATCH,), jnp.float32),
        scratch_types=[
            pltpu.VMEM((_NCHUNK, _CHUNK), jnp.int32),
            pltpu.VMEM((_NCHUNK, _CHUNK), jnp.int32),
            pltpu.VMEM((_BPW, EMBED_DIM), jnp.float32),
            pltpu.VMEM((_BPW, EMBED_DIM), jnp.float32),
            pltpu.VMEM((16, 16), jnp.float32),
            pltpu.VMEM((_BPW,), jnp.float32),
            pltpu.SemaphoreType.DMA,
        ],
    )(_sgns_body)
    return run(c, w, c_embeds, w_embeds)


def kernel(c, w, c_embeds, w_embeds):
    return _sgns(c.astype(jnp.int32), w.astype(jnp.int32), c_embeds, w_embeds)
